# baseline (speedup 1.0000x reference)
import jax
import jax.numpy as jnp
from jax.experimental import pallas as pl
from jax.experimental.pallas import tpu as pltpu

_T = 4.0
_ALPHA = 1.0
_BETA = 8.0
_K = 100
_C = 1000
_R = 512

_Z90 = 1.2815516
_INV_SLOPE = 1.0 / 175.498
_EMAX = 6
_IMIN = -2147483647 - 1
_IMAX = 2147483647


def _sortable(v):
    b = jax.lax.bitcast_convert_type(v, jnp.int32)
    return b ^ (jax.lax.shift_right_arithmetic(b, 31) & jnp.int32(0x7FFFFFFF))


def _dkd_block(s_ref, t_ref, g_ref, out_ref, lo_ref, hi_ref, cl_ref, ch_ref,
               pb_ref, er_ref, fl_ref, xo_ref, m_ref, flag_ref):
    sraw = s_ref[...]
    traw = t_ref[...]
    g = g_ref[0, 0, :]
    rows = sraw.shape[0]

    s = sraw * jnp.float32(1.0 / _T)
    t = traw * jnp.float32(1.0 / _T)

    key = _sortable(jnp.where(traw == 0.0, jnp.float32(0.0), traw))

    def _probe(lo, hi, cl, ch, mid):
        mid = jnp.minimum(jnp.maximum(mid, lo + 1), hi)
        cnt = jnp.sum((key >= mid).astype(jnp.int32), axis=1, keepdims=True)
        ge = cnt >= _K
        lo = jnp.where(ge, mid, lo)
        cl = jnp.where(ge, cnt, cl)
        hi = jnp.where(ge, hi, mid - 1)
        ch = jnp.where(ge, ch, cnt)
        return lo, hi, cl, ch, cnt

    def _fp_mid(lo, hi, cl, ch):
        frac = (cl - _K).astype(jnp.float32) / (cl - ch).astype(jnp.float32)
        flo = lo.astype(jnp.float32)
        return lo + (frac * (hi.astype(jnp.float32) - flo)).astype(jnp.int32)

    lo = jnp.full((rows, 1), jnp.int32(_IMIN), jnp.int32)
    hi = jnp.full((rows, 1), jnp.int32(_IMAX), jnp.int32)
    cl = jnp.full((rows, 1), jnp.int32(_C), jnp.int32)
    ch = jnp.zeros((rows, 1), jnp.int32)

    p = jnp.full((rows, 1), jnp.float32(_Z90), jnp.float32)
    lo, hi, cl, ch, c1 = _probe(lo, hi, cl, ch, _sortable(p))
    p = p + (c1 - _K).astype(jnp.float32) * jnp.float32(_INV_SLOPE)
    lo, hi, cl, ch, c2 = _probe(lo, hi, cl, ch, _sortable(p))
    p = p + (c2 - _K).astype(jnp.float32) * jnp.float32(_INV_SLOPE)
    lo, hi, cl, ch, _ = _probe(lo, hi, cl, ch, _sortable(p))
    lo, hi, cl, ch, _ = _probe(lo, hi, cl, ch, _fp_mid(lo, hi, cl, ch))
    lo, hi, cl, ch, _ = _probe(lo, hi, cl, ch, _fp_mid(lo, hi, cl, ch))

    lo_ref[...] = lo
    hi_ref[...] = hi
    cl_ref[...] = cl
    ch_ref[...] = ch
    mind = jnp.minimum(cl - _K, _K - ch)
    flag_ref[0] = jnp.logical_not(
        jnp.all((mind <= _EMAX) | (lo >= hi))).astype(jnp.int32)

    for _ in range(8):
        @pl.when(flag_ref[0] == 1)
        def _bchunk():
            blo, bhi = lo_ref[...], hi_ref[...]
            bcl, bch = cl_ref[...], ch_ref[...]
            for _ in range(4):
                bmid = (blo >> 1) + (bhi >> 1) + ((blo | bhi) & 1)
                blo, bhi, bcl, bch, _ = _probe(blo, bhi, bcl, bch, bmid)
            blo, bhi, bcl, bch, _ = _probe(blo, bhi, bcl, bch,
                                           _fp_mid(blo, bhi, bcl, bch))
            lo_ref[...] = blo
            hi_ref[...] = bhi
            cl_ref[...] = bcl
            ch_ref[...] = bch
            bmind = jnp.minimum(bcl - _K, _K - bch)
            flag_ref[0] = jnp.logical_not(
                jnp.all((bmind <= _EMAX) | (blo >= bhi))).astype(jnp.int32)

    lo = lo_ref[...]
    hi = hi_ref[...]
    cl = cl_ref[...]
    ch = ch_ref[...]
    e = cl - _K
    d = _K - ch
    tie = (lo >= hi) & (cl != _K)
    flip = d < e
    er0 = jnp.where(tie, jnp.int32(0), jnp.minimum(e, d))
    fl_ref[...] = flip.astype(jnp.int32)
    m_ref[...] = jnp.full((rows, 1), jnp.int32(-1), jnp.int32)
    xo_ref[...] = jnp.zeros((rows, 1), jnp.int32)
    col = jax.lax.broadcasted_iota(jnp.int32, (rows, _C), 1)

    def _index_cutoff(eqm, kc):
        ilo = jnp.zeros((rows, 1), jnp.int32)
        ihi = jnp.full((rows, 1), jnp.int32(_C - 1), jnp.int32)
        for _ in range(10):
            imid = (ilo + ihi) >> 1
            cc = jnp.sum((eqm & (col <= imid)).astype(jnp.int32), axis=1,
                         keepdims=True)
            take = cc >= kc
            ihi = jnp.where(take, imid, ihi)
            ilo = jnp.where(take, ilo, imid + 1)
        return ilo

    def _peel_steps(flip_i, pb, er, steps):
        fsel = jnp.where(flip_i, jnp.int32(-1), jnp.int32(0))
        pkey = key ^ fsel
        for _ in range(steps):
            rmask = pkey >= pb
            mn = jnp.min(jnp.where(rmask, pkey, jnp.int32(_IMAX)), axis=1,
                         keepdims=True)
            eqm = rmask & (pkey == mn)
            cmn = jnp.sum(eqm.astype(jnp.int32), axis=1, keepdims=True)
            act = er > 0
            full = act & (cmn <= er)
            partial = act & (cmn > er)
            pb = jnp.where(full, mn + 1, pb)
            er = jnp.where(full, er - cmn, er)

            @pl.when(jnp.any(partial))
            def _partial():
                kc = jnp.where(flip_i, er, cmn - er)
                mcut = _index_cutoff(eqm & partial, kc)
                m_ref[...] = jnp.where(partial, mcut, m_ref[...])
                xo_ref[...] = jnp.where(
                    partial, jnp.where(flip_i, ~mn, mn), xo_ref[...])

            er = jnp.where(partial, jnp.int32(0), er)
        return pb, er

    pb1, er1 = _peel_steps(flip, jnp.where(flip, ~hi, lo), er0, 2)
    pb_ref[...] = pb1
    er_ref[...] = er1
    flag_ref[1] = jnp.any(er1 > 0).astype(jnp.int32)
    for _ in range(2):
        @pl.when(flag_ref[1] == 1)
        def _pchunk():
            flip_i = fl_ref[...] != 0
            pb, er = _peel_steps(flip_i, pb_ref[...], er_ref[...], 2)
            pb_ref[...] = pb
            er_ref[...] = er
            flag_ref[1] = jnp.any(er > 0).astype(jnp.int32)

    @pl.when(jnp.any(tie))
    def _tie_path():
        n_hi = jnp.sum((key > lo).astype(jnp.int32), axis=1, keepdims=True)
        eqt = (key == lo) & tie
        mcut = _index_cutoff(eqt, jnp.int32(_K) - n_hi)
        m_ref[...] = jnp.where(tie, mcut, m_ref[...])
        xo_ref[...] = jnp.where(tie, lo, xo_ref[...])

    m = m_ref[...]
    pb = pb_ref[...]
    flip_v = fl_ref[...] != 0
    x = jnp.where(m >= 0, xo_ref[...], jnp.where(flip_v, ~pb, pb - 1))
    topk_mask = (key > x) | ((key == x) & (col <= m))

    gtm = col == g[:, None]
    other = topk_mask & jnp.logical_not(gtm)

    ms = jnp.max(s, axis=1, keepdims=True)
    mt = jnp.max(t, axis=1, keepdims=True)
    es = jnp.exp(s - ms)
    et = jnp.exp(t - mt)
    zs = jnp.sum(es, axis=1, keepdims=True)
    zt = jnp.sum(et, axis=1, keepdims=True)
    lzs = jnp.log(zs)
    lzt = jnp.log(zt)
    s_g = jnp.sum(jnp.where(gtm, s, 0.0), axis=1, keepdims=True)
    t_g = jnp.sum(jnp.where(gtm, t, 0.0), axis=1, keepdims=True)
    lps1 = s_g - ms - lzs
    lpt1 = t_g - mt - lzt
    ps2n = jnp.sum(jnp.where(other, es, 0.0), axis=1, keepdims=True)
    pt2n = jnp.sum(jnp.where(other, et, 0.0), axis=1, keepdims=True)
    lps2n = jnp.log(ps2n)
    lpt2n = jnp.log(pt2n)
    tckd = (jnp.exp(lpt1) * (lpt1 - lps1)
            + pt2n / zt * ((lpt2n - lzt) - (lps2n - lzs)))

    kl_num = jnp.sum(jnp.where(other, et * (t - s), 0.0), axis=1,
                     keepdims=True)
    nckd = kl_num / pt2n + (ms + lps2n) - (mt + lpt2n)

    out_ref[0, :, :] = jnp.sum(_ALPHA * tckd + _BETA * nckd, axis=0,
                               keepdims=True)


@jax.jit
def kernel(logits_student, logits_teacher, target):
    bsz, c = logits_teacher.shape
    nblk = bsz // _R
    tgt = target.reshape(nblk, 1, _R)
    out = pl.pallas_call(
        _dkd_block,
        grid=(nblk,),
        in_specs=[
            pl.BlockSpec((_R, c), lambda i: (i, 0)),
            pl.BlockSpec((_R, c), lambda i: (i, 0)),
            pl.BlockSpec((1, 1, _R), lambda i: (i, 0, 0)),
        ],
        out_specs=pl.BlockSpec((1, 1, 1), lambda i: (i, 0, 0)),
        out_shape=jax.ShapeDtypeStruct((nblk, 1, 1), jnp.float32),
        scratch_shapes=[pltpu.VMEM((_R, 1), jnp.int32) for _ in range(9)]
        + [pltpu.SMEM((2,), jnp.int32)],
        compiler_params=pltpu.CompilerParams(
            dimension_semantics=("parallel",),
        ),
    )(logits_student, logits_teacher, tgt)
    return jnp.sum(out) * jnp.float32(_T * _T / bsz)


# speedup vs baseline: 3.2456x; 3.2456x over previous
import jax
import jax.numpy as jnp
from jax.experimental import pallas as pl
from jax.experimental.pallas import tpu as pltpu

_T = 4.0
_ALPHA = 1.0
_BETA = 8.0
_K = 100
_C = 1000
_R = 256

_Z90 = 1.2815516
_INV_SLOPE = 1.0 / 175.498
_EMAX = 6
_IMIN = -2147483647 - 1
_IMAX = 2147483647


def _sortable(v):
    b = jax.lax.bitcast_convert_type(v, jnp.int32)
    return b ^ (jax.lax.shift_right_arithmetic(b, 31) & jnp.int32(0x7FFFFFFF))


def _dkd_block(s_ref, t_ref, g_ref, out_ref, lo_ref, hi_ref, cl_ref, ch_ref,
               pb_ref, er_ref, fl_ref, xo_ref, m_ref, flag_ref):
    sraw = s_ref[...]
    traw = t_ref[...]
    g = g_ref[0, 0, :]
    rows = sraw.shape[0]

    s = sraw * jnp.float32(1.0 / _T)
    t = traw * jnp.float32(1.0 / _T)

    key = _sortable(jnp.where(traw == 0.0, jnp.float32(0.0), traw))

    def _probe(lo, hi, cl, ch, mid):
        mid = jnp.minimum(jnp.maximum(mid, lo + 1), hi)
        cnt = jnp.sum((key >= mid).astype(jnp.int32), axis=1, keepdims=True)
        ge = cnt >= _K
        lo = jnp.where(ge, mid, lo)
        cl = jnp.where(ge, cnt, cl)
        hi = jnp.where(ge, hi, mid - 1)
        ch = jnp.where(ge, ch, cnt)
        return lo, hi, cl, ch, cnt

    def _fp_mid(lo, hi, cl, ch):
        frac = (cl - _K).astype(jnp.float32) / (cl - ch).astype(jnp.float32)
        flo = lo.astype(jnp.float32)
        return lo + (frac * (hi.astype(jnp.float32) - flo)).astype(jnp.int32)

    lo = jnp.full((rows, 1), jnp.int32(_IMIN), jnp.int32)
    hi = jnp.full((rows, 1), jnp.int32(_IMAX), jnp.int32)
    cl = jnp.full((rows, 1), jnp.int32(_C), jnp.int32)
    ch = jnp.zeros((rows, 1), jnp.int32)

    p = jnp.full((rows, 1), jnp.float32(_Z90), jnp.float32)
    lo, hi, cl, ch, c1 = _probe(lo, hi, cl, ch, _sortable(p))
    p = p + (c1 - _K).astype(jnp.float32) * jnp.float32(_INV_SLOPE)
    lo, hi, cl, ch, c2 = _probe(lo, hi, cl, ch, _sortable(p))
    p = p + (c2 - _K).astype(jnp.float32) * jnp.float32(_INV_SLOPE)
    lo, hi, cl, ch, _ = _probe(lo, hi, cl, ch, _sortable(p))
    lo, hi, cl, ch, _ = _probe(lo, hi, cl, ch, _fp_mid(lo, hi, cl, ch))
    lo, hi, cl, ch, _ = _probe(lo, hi, cl, ch, _fp_mid(lo, hi, cl, ch))

    lo_ref[...] = lo
    hi_ref[...] = hi
    cl_ref[...] = cl
    ch_ref[...] = ch
    mind = jnp.minimum(cl - _K, _K - ch)
    flag_ref[0] = jnp.logical_not(
        jnp.all((mind <= _EMAX) | (lo >= hi))).astype(jnp.int32)

    for _ in range(8):
        @pl.when(flag_ref[0] == 1)
        def _bchunk():
            blo, bhi = lo_ref[...], hi_ref[...]
            bcl, bch = cl_ref[...], ch_ref[...]
            for _ in range(4):
                bmid = (blo >> 1) + (bhi >> 1) + ((blo | bhi) & 1)
                blo, bhi, bcl, bch, _ = _probe(blo, bhi, bcl, bch, bmid)
            blo, bhi, bcl, bch, _ = _probe(blo, bhi, bcl, bch,
                                           _fp_mid(blo, bhi, bcl, bch))
            lo_ref[...] = blo
            hi_ref[...] = bhi
            cl_ref[...] = bcl
            ch_ref[...] = bch
            bmind = jnp.minimum(bcl - _K, _K - bch)
            flag_ref[0] = jnp.logical_not(
                jnp.all((bmind <= _EMAX) | (blo >= bhi))).astype(jnp.int32)

    lo = lo_ref[...]
    hi = hi_ref[...]
    cl = cl_ref[...]
    ch = ch_ref[...]
    e = cl - _K
    d = _K - ch
    tie = (lo >= hi) & (cl != _K)
    flip = d < e
    er0 = jnp.where(tie, jnp.int32(0), jnp.minimum(e, d))
    fl_ref[...] = flip.astype(jnp.int32)
    m_ref[...] = jnp.full((rows, 1), jnp.int32(-1), jnp.int32)
    xo_ref[...] = jnp.zeros((rows, 1), jnp.int32)
    col = jax.lax.broadcasted_iota(jnp.int32, (rows, _C), 1)

    def _index_cutoff(eqm, kc):
        ilo = jnp.zeros((rows, 1), jnp.int32)
        ihi = jnp.full((rows, 1), jnp.int32(_C - 1), jnp.int32)
        for _ in range(10):
            imid = (ilo + ihi) >> 1
            cc = jnp.sum((eqm & (col <= imid)).astype(jnp.int32), axis=1,
                         keepdims=True)
            take = cc >= kc
            ihi = jnp.where(take, imid, ihi)
            ilo = jnp.where(take, ilo, imid + 1)
        return ilo

    def _peel_steps(flip_i, pb, er, steps):
        fsel = jnp.where(flip_i, jnp.int32(-1), jnp.int32(0))
        pkey = key ^ fsel
        for _ in range(steps):
            rmask = pkey >= pb
            mn = jnp.min(jnp.where(rmask, pkey, jnp.int32(_IMAX)), axis=1,
                         keepdims=True)
            eqm = rmask & (pkey == mn)
            cmn = jnp.sum(eqm.astype(jnp.int32), axis=1, keepdims=True)
            act = er > 0
            full = act & (cmn <= er)
            partial = act & (cmn > er)
            pb = jnp.where(full, mn + 1, pb)
            er = jnp.where(full, er - cmn, er)

            @pl.when(jnp.any(partial))
            def _partial():
                kc = jnp.where(flip_i, er, cmn - er)
                mcut = _index_cutoff(eqm & partial, kc)
                m_ref[...] = jnp.where(partial, mcut, m_ref[...])
                xo_ref[...] = jnp.where(
                    partial, jnp.where(flip_i, ~mn, mn), xo_ref[...])

            er = jnp.where(partial, jnp.int32(0), er)
        return pb, er

    pb1, er1 = _peel_steps(flip, jnp.where(flip, ~hi, lo), er0, 2)
    pb_ref[...] = pb1
    er_ref[...] = er1
    flag_ref[1] = jnp.any(er1 > 0).astype(jnp.int32)
    for _ in range(2):
        @pl.when(flag_ref[1] == 1)
        def _pchunk():
            flip_i = fl_ref[...] != 0
            pb, er = _peel_steps(flip_i, pb_ref[...], er_ref[...], 2)
            pb_ref[...] = pb
            er_ref[...] = er
            flag_ref[1] = jnp.any(er > 0).astype(jnp.int32)

    @pl.when(jnp.any(tie))
    def _tie_path():
        n_hi = jnp.sum((key > lo).astype(jnp.int32), axis=1, keepdims=True)
        eqt = (key == lo) & tie
        mcut = _index_cutoff(eqt, jnp.int32(_K) - n_hi)
        m_ref[...] = jnp.where(tie, mcut, m_ref[...])
        xo_ref[...] = jnp.where(tie, lo, xo_ref[...])

    m = m_ref[...]
    pb = pb_ref[...]
    flip_v = fl_ref[...] != 0
    x = jnp.where(m >= 0, xo_ref[...], jnp.where(flip_v, ~pb, pb - 1))
    topk_mask = (key > x) | ((key == x) & (col <= m))

    gtm = col == g[:, None]
    other = topk_mask & jnp.logical_not(gtm)

    ms = jnp.max(s, axis=1, keepdims=True)
    mt = jnp.max(t, axis=1, keepdims=True)
    es = jnp.exp(s - ms)
    et = jnp.exp(t - mt)
    zs = jnp.sum(es, axis=1, keepdims=True)
    zt = jnp.sum(et, axis=1, keepdims=True)
    lzs = jnp.log(zs)
    lzt = jnp.log(zt)
    s_g = jnp.sum(jnp.where(gtm, s, 0.0), axis=1, keepdims=True)
    t_g = jnp.sum(jnp.where(gtm, t, 0.0), axis=1, keepdims=True)
    lps1 = s_g - ms - lzs
    lpt1 = t_g - mt - lzt
    ps2n = jnp.sum(jnp.where(other, es, 0.0), axis=1, keepdims=True)
    pt2n = jnp.sum(jnp.where(other, et, 0.0), axis=1, keepdims=True)
    lps2n = jnp.log(ps2n)
    lpt2n = jnp.log(pt2n)
    tckd = (jnp.exp(lpt1) * (lpt1 - lps1)
            + pt2n / zt * ((lpt2n - lzt) - (lps2n - lzs)))

    kl_num = jnp.sum(jnp.where(other, et * (t - s), 0.0), axis=1,
                     keepdims=True)
    nckd = kl_num / pt2n + (ms + lps2n) - (mt + lpt2n)

    out_ref[0, :, :] = jnp.sum(_ALPHA * tckd + _BETA * nckd, axis=0,
                               keepdims=True)


@jax.jit
def kernel(logits_student, logits_teacher, target):
    bsz, c = logits_teacher.shape
    nblk = bsz // _R
    tgt = target.reshape(nblk, 1, _R)
    out = pl.pallas_call(
        _dkd_block,
        grid=(nblk,),
        in_specs=[
            pl.BlockSpec((_R, c), lambda i: (i, 0)),
            pl.BlockSpec((_R, c), lambda i: (i, 0)),
            pl.BlockSpec((1, 1, _R), lambda i: (i, 0, 0)),
        ],
        out_specs=pl.BlockSpec((1, 1, 1), lambda i: (i, 0, 0)),
        out_shape=jax.ShapeDtypeStruct((nblk, 1, 1), jnp.float32),
        scratch_shapes=[pltpu.VMEM((_R, 1), jnp.int32) for _ in range(9)]
        + [pltpu.SMEM((2,), jnp.int32)],
        compiler_params=pltpu.CompilerParams(
            dimension_semantics=("parallel",),
        ),
    )(logits_student, logits_teacher, tgt)
    return jnp.sum(out) * jnp.float32(_T * _T / bsz)
